# baseline (speedup 1.0000x reference)
import functools

import jax
import jax.numpy as jnp
from jax import lax
from jax.experimental import pallas as pl
from jax.experimental.pallas import tpu as pltpu
from jax.experimental.pallas import tpu_sc as plsc

N = 10000
E = 320000
D_IN = 128
D_OUT = 128

NC = 2
NS = 16
NW = NC * NS
EPW = E // NW
CHUNK = 80
NCHUNK = EPW // CHUNK
N_PAD = 10240
ROWS_PER_TILE = N_PAD // NS
ZROWS = 16
NBUF = 3


def _matmul_body(x_ref, w_ref, o_ref):
    o_ref[...] = jnp.dot(x_ref[...], w_ref[...],
                         preferred_element_type=jnp.float32)


def _matmul(x, W):
    return pl.pallas_call(
        _matmul_body,
        out_shape=jax.ShapeDtypeStruct((N, D_OUT), jnp.float32),
        grid=(10,),
        in_specs=[
            pl.BlockSpec((N // 10, D_IN), lambda i: (i, 0)),
            pl.BlockSpec((D_IN, D_OUT), lambda i: (0, 0)),
        ],
        out_specs=pl.BlockSpec((N // 10, D_OUT), lambda i: (i, 0)),
    )(x, W)


def _sc_body(pre_hbm, src_hbm, dst_hbm, ew_hbm, part_hbm,
             accum, src_all, dst_idx0, dst_idx1, dst_idx2,
             wts0, wts1, wts2, rows0, rows1, rows2, zbuf,
             gsem0, gsem1, gsem2, dsem0, dsem1, dsem2, ssem0, ssem1, ssem2):
    cid = lax.axis_index("c")
    sid = lax.axis_index("s")
    wid = sid * NC + cid
    ebase = wid * EPW

    dst_idx = [dst_idx0, dst_idx1, dst_idx2]
    wts = [wts0, wts1, wts2]
    rows = [rows0, rows1, rows2]
    gsem = [gsem0, gsem1, gsem2]
    dsem = [dsem0, dsem1, dsem2]
    ssem = [ssem0, ssem1, ssem2]

    hoist = pltpu.async_copy(src_hbm.at[pl.ds(ebase, EPW)], src_all, gsem0)

    def _zero_row(i, _):
        for j in range(D_OUT // 16):
            zbuf[i, pl.ds(j * 16, 16)] = jnp.zeros((16,), jnp.float32)
        return _
    lax.fori_loop(0, ZROWS, _zero_row, 0)
    for z in range(ROWS_PER_TILE // ZROWS):
        pltpu.sync_copy(
            zbuf, accum.at[pl.ds(sid * ROWS_PER_TILE + z * ZROWS, ZROWS)])
    hoist.wait()
    plsc.subcore_barrier()

    def _fire(c, b):
        pltpu.async_copy(dst_hbm.at[pl.ds(ebase + c * CHUNK, CHUNK)],
                         dst_idx[b], dsem[b])
        pltpu.async_copy(ew_hbm.at[pl.ds(ebase + c * CHUNK, CHUNK)],
                         wts[b], dsem[b])
        pltpu.async_copy(pre_hbm.at[src_all.at[pl.ds(c * CHUNK, CHUNK)]],
                         rows[b], gsem[b])

    def _wait_scatter(b):
        pltpu.make_async_copy(rows[b], accum.at[dst_idx[b]], ssem[b]).wait()

    def _step(c, b, first=False):
        pltpu.make_async_copy(dst_hbm.at[pl.ds(0, CHUNK)],
                              dst_idx[b], dsem[b]).wait()
        pltpu.make_async_copy(ew_hbm.at[pl.ds(0, CHUNK)],
                              wts[b], dsem[b]).wait()
        pltpu.make_async_copy(pre_hbm.at[pl.ds(0, CHUNK)],
                              rows[b], gsem[b]).wait()

        b2 = (b + 2) % NBUF
        if not isinstance(c, int) or c + 2 < NCHUNK:
            if not first:
                _wait_scatter(b2)
            _fire(c + 2, b2)

        def _scale(k16, _2):
            base = k16 * 16
            ws = [plsc.load_gather(wts[b],
                                   [jnp.full((16,), base + u, jnp.int32)])
                  for u in range(16)]
            for u in range(16):
                k = base + u
                for j in range(D_OUT // 16):
                    rows[b][k, pl.ds(j * 16, 16)] = (
                        rows[b][k, pl.ds(j * 16, 16)] * ws[u])
            return _2
        lax.fori_loop(0, CHUNK // 16, _scale, 0)
        pltpu.async_copy(rows[b], accum.at[dst_idx[b]], ssem[b], add=True)

    _fire(0, 0)
    _fire(1, 1)
    _step(0, 0, first=True)

    def _trip(i, _):
        c = 3 * i + 1
        _step(c, 1)
        _step(c + 1, 2)
        _step(c + 2, 0)
        return _
    NTRIP = (NCHUNK - 5) // 3
    lax.fori_loop(0, NTRIP, _trip, 0)

    for c in range(3 * NTRIP + 1, NCHUNK):
        _step(c, c % NBUF)
    for b in range(NBUF):
        _wait_scatter(b)
    plsc.subcore_barrier()

    pltpu.sync_copy(
        accum.at[pl.ds(sid * ROWS_PER_TILE, ROWS_PER_TILE)],
        part_hbm.at[cid, pl.ds(sid * ROWS_PER_TILE, ROWS_PER_TILE)])


_sc_gcn = functools.partial(
    pl.kernel,
    out_type=jax.ShapeDtypeStruct((NC, N_PAD, D_OUT), jnp.float32),
    mesh=plsc.VectorSubcoreMesh(core_axis_name="c", subcore_axis_name="s"),
    scratch_types=(
        [pltpu.VMEM_SHARED((N_PAD, D_OUT), jnp.float32)]
        + [pltpu.VMEM((EPW,), jnp.int32)]
        + [pltpu.VMEM((CHUNK,), jnp.int32)] * NBUF
        + [pltpu.VMEM((CHUNK,), jnp.float32)] * NBUF
        + [pltpu.VMEM((CHUNK, D_OUT), jnp.float32)] * NBUF
        + [pltpu.VMEM((ZROWS, D_OUT), jnp.float32)]
        + [pltpu.SemaphoreType.DMA] * (3 * NBUF)
    ),
    compiler_params=pltpu.CompilerParams(needs_layout_passes=False),
)(_sc_body)


def _combine_body(p_ref, o_ref):
    o_ref[...] = jnp.maximum(p_ref[0] + p_ref[1], 0.0)


def _combine(part):
    return pl.pallas_call(
        _combine_body,
        out_shape=jax.ShapeDtypeStruct((N, D_OUT), jnp.float32),
        grid=(10,),
        in_specs=[pl.BlockSpec((NC, N // 10, D_OUT), lambda i: (0, i, 0))],
        out_specs=pl.BlockSpec((N // 10, D_OUT), lambda i: (i, 0)),
    )(part)


def kernel(x, W, edge_weight, edge_index):
    src = edge_index[0].astype(jnp.int32)
    dst = edge_index[1].astype(jnp.int32)
    pre_sup = _matmul(x, W)
    part = _sc_gcn(pre_sup, src, dst, edge_weight)
    return _combine(part)


# speedup vs baseline: 1.1225x; 1.1225x over previous
import functools

import jax
import jax.numpy as jnp
from jax import lax
from jax.experimental import pallas as pl
from jax.experimental.pallas import tpu as pltpu
from jax.experimental.pallas import tpu_sc as plsc

N = 10000
E = 320000
D_IN = 128
D_OUT = 128

NC = 2
NS = 16
NW = NC * NS
EPW = E // NW
CHUNK = 80
NCHUNK = EPW // CHUNK
N_PAD = 10240
ROWS_PER_TILE = N_PAD // NS
ZROWS = 16
NBUF = 3


def _matmul_body(x_ref, w_ref, o_ref):
    o_ref[...] = jnp.dot(x_ref[...], w_ref[...],
                         preferred_element_type=jnp.float32)


def _matmul(x, W):
    return pl.pallas_call(
        _matmul_body,
        out_shape=jax.ShapeDtypeStruct((N, D_OUT), jnp.float32),
        grid=(10,),
        in_specs=[
            pl.BlockSpec((N // 10, D_IN), lambda i: (i, 0)),
            pl.BlockSpec((D_IN, D_OUT), lambda i: (0, 0)),
        ],
        out_specs=pl.BlockSpec((N // 10, D_OUT), lambda i: (i, 0)),
    )(x, W)


def _sc_body(pre_hbm, src_hbm, dst_hbm, ew_hbm, part_hbm,
             accum, src_all, dst_idx0, dst_idx1, dst_idx2,
             wts0, wts1, wts2, rows0, rows1, rows2, zbuf,
             gsem0, gsem1, gsem2, dsem0, dsem1, dsem2, ssem0, ssem1, ssem2):
    cid = lax.axis_index("c")
    sid = lax.axis_index("s")
    wid = sid * NC + cid
    ebase = wid * EPW

    dst_idx = [dst_idx0, dst_idx1, dst_idx2]
    wts = [wts0, wts1, wts2]
    rows = [rows0, rows1, rows2]
    gsem = [gsem0, gsem1, gsem2]
    dsem = [dsem0, dsem1, dsem2]
    ssem = [ssem0, ssem1, ssem2]

    hoist = pltpu.async_copy(src_hbm.at[pl.ds(ebase, EPW)], src_all, gsem0)

    def _zero_row(i, _):
        for j in range(D_OUT // 16):
            zbuf[i, pl.ds(j * 16, 16)] = jnp.zeros((16,), jnp.float32)
        return _
    lax.fori_loop(0, ZROWS, _zero_row, 0)
    for z in range(ROWS_PER_TILE // ZROWS):
        pltpu.sync_copy(
            zbuf, accum.at[pl.ds(sid * ROWS_PER_TILE + z * ZROWS, ZROWS)])
    hoist.wait()
    plsc.subcore_barrier()

    def _fire(c, b):
        pltpu.async_copy(dst_hbm.at[pl.ds(ebase + c * CHUNK, CHUNK)],
                         dst_idx[b], dsem[b])
        pltpu.async_copy(ew_hbm.at[pl.ds(ebase + c * CHUNK, CHUNK)],
                         wts[b], dsem[b])
        pltpu.async_copy(pre_hbm.at[src_all.at[pl.ds(c * CHUNK, CHUNK)]],
                         rows[b], gsem[b])

    def _wait_scatter(b):
        pltpu.make_async_copy(rows[b], accum.at[dst_idx[b]], ssem[b]).wait()

    def _step(c, b, first=False):
        pltpu.make_async_copy(dst_hbm.at[pl.ds(0, CHUNK)],
                              dst_idx[b], dsem[b]).wait()
        pltpu.make_async_copy(ew_hbm.at[pl.ds(0, CHUNK)],
                              wts[b], dsem[b]).wait()
        pltpu.make_async_copy(pre_hbm.at[pl.ds(0, CHUNK)],
                              rows[b], gsem[b]).wait()

        def _scale(k16, _2):
            base = k16 * 16
            ws = [plsc.load_gather(wts[b],
                                   [jnp.full((16,), base + u, jnp.int32)])
                  for u in range(16)]
            for u in range(16):
                k = base + u
                for j in range(D_OUT // 16):
                    rows[b][k, pl.ds(j * 16, 16)] = (
                        rows[b][k, pl.ds(j * 16, 16)] * ws[u])
            return _2
        lax.fori_loop(0, CHUNK // 16, _scale, 0)
        pltpu.async_copy(rows[b], accum.at[dst_idx[b]], ssem[b], add=True)

        b2 = (b + 2) % NBUF
        if not isinstance(c, int) or c + 2 < NCHUNK:
            if not first:
                _wait_scatter(b2)
            _fire(c + 2, b2)

    _fire(0, 0)
    _fire(1, 1)
    _step(0, 0, first=True)

    def _trip(i, _):
        c = 3 * i + 1
        _step(c, 1)
        _step(c + 1, 2)
        _step(c + 2, 0)
        return _
    NTRIP = (NCHUNK - 5) // 3
    lax.fori_loop(0, NTRIP, _trip, 0)

    for c in range(3 * NTRIP + 1, NCHUNK):
        _step(c, c % NBUF)
    for b in range(NBUF):
        _wait_scatter(b)
    plsc.subcore_barrier()

    pltpu.sync_copy(
        accum.at[pl.ds(sid * ROWS_PER_TILE, ROWS_PER_TILE)],
        part_hbm.at[cid, pl.ds(sid * ROWS_PER_TILE, ROWS_PER_TILE)])


_sc_gcn = functools.partial(
    pl.kernel,
    out_type=jax.ShapeDtypeStruct((NC, N_PAD, D_OUT), jnp.float32),
    mesh=plsc.VectorSubcoreMesh(core_axis_name="c", subcore_axis_name="s"),
    scratch_types=(
        [pltpu.VMEM_SHARED((N_PAD, D_OUT), jnp.float32)]
        + [pltpu.VMEM((EPW,), jnp.int32)]
        + [pltpu.VMEM((CHUNK,), jnp.int32)] * NBUF
        + [pltpu.VMEM((CHUNK,), jnp.float32)] * NBUF
        + [pltpu.VMEM((CHUNK, D_OUT), jnp.float32)] * NBUF
        + [pltpu.VMEM((ZROWS, D_OUT), jnp.float32)]
        + [pltpu.SemaphoreType.DMA] * (3 * NBUF)
    ),
    compiler_params=pltpu.CompilerParams(needs_layout_passes=False),
)(_sc_body)


def _combine_body(p_ref, o_ref):
    o_ref[...] = jnp.maximum(p_ref[0] + p_ref[1], 0.0)


def _combine(part):
    return pl.pallas_call(
        _combine_body,
        out_shape=jax.ShapeDtypeStruct((N, D_OUT), jnp.float32),
        grid=(10,),
        in_specs=[pl.BlockSpec((NC, N // 10, D_OUT), lambda i: (0, i, 0))],
        out_specs=pl.BlockSpec((N // 10, D_OUT), lambda i: (i, 0)),
    )(part)


def kernel(x, W, edge_weight, edge_index):
    src = edge_index[0].astype(jnp.int32)
    dst = edge_index[1].astype(jnp.int32)
    pre_sup = _matmul(x, W)
    part = _sc_gcn(pre_sup, src, dst, edge_weight)
    return _combine(part)


# speedup vs baseline: 1.2269x; 1.0930x over previous
import functools

import jax
import jax.numpy as jnp
from jax import lax
from jax.experimental import pallas as pl
from jax.experimental.pallas import tpu as pltpu
from jax.experimental.pallas import tpu_sc as plsc

N = 10000
E = 320000
D_IN = 128
D_OUT = 128

NC = 2
NS = 16
NW = NC * NS
EPW = E // NW
CHUNK = 80
NCHUNK = EPW // CHUNK
N_PAD = 10240
ROWS_PER_TILE = N_PAD // NS
ZROWS = 16
NBUF = 3


def _matmul_body(x_ref, w_ref, o_ref):
    o_ref[...] = jnp.dot(x_ref[...], w_ref[...],
                         preferred_element_type=jnp.float32)


def _matmul(x, W):
    return pl.pallas_call(
        _matmul_body,
        out_shape=jax.ShapeDtypeStruct((N, D_OUT), jnp.float32),
        grid=(10,),
        in_specs=[
            pl.BlockSpec((N // 10, D_IN), lambda i: (i, 0)),
            pl.BlockSpec((D_IN, D_OUT), lambda i: (0, 0)),
        ],
        out_specs=pl.BlockSpec((N // 10, D_OUT), lambda i: (i, 0)),
    )(x, W)


def _sc_body(pre_hbm, src_hbm, dst_hbm, ew_hbm, part_hbm,
             accum, src_all, dst_idx0, dst_idx1, dst_idx2,
             wts0, wts1, wts2, rows0, rows1, rows2, zbuf,
             gsem0, gsem1, gsem2, dsem0, dsem1, dsem2, ssem0, ssem1, ssem2):
    cid = lax.axis_index("c")
    sid = lax.axis_index("s")
    wid = sid * NC + cid
    ebase = wid * EPW

    dst_idx = [dst_idx0, dst_idx1, dst_idx2]
    wts = [wts0, wts1, wts2]
    rows = [rows0, rows1, rows2]
    gsem = [gsem0, gsem1, gsem2]
    dsem = [dsem0, dsem1, dsem2]
    ssem = [ssem0, ssem1, ssem2]

    hoist = pltpu.async_copy(src_hbm.at[pl.ds(ebase, EPW)], src_all, gsem0)

    def _zero_row(i, _):
        for j in range(D_OUT // 16):
            zbuf[i, pl.ds(j * 16, 16)] = jnp.zeros((16,), jnp.float32)
        return _
    lax.fori_loop(0, ZROWS, _zero_row, 0)
    for z in range(ROWS_PER_TILE // ZROWS):
        pltpu.sync_copy(
            zbuf, accum.at[pl.ds(sid * ROWS_PER_TILE + z * ZROWS, ZROWS)])
    hoist.wait()
    plsc.subcore_barrier()

    def _fire(c, b):
        pltpu.async_copy(dst_hbm.at[pl.ds(ebase + c * CHUNK, CHUNK)],
                         dst_idx[b], dsem[b])
        pltpu.async_copy(ew_hbm.at[pl.ds(ebase + c * CHUNK, CHUNK)],
                         wts[b], dsem[b])
        pltpu.async_copy(pre_hbm.at[src_all.at[pl.ds(c * CHUNK, CHUNK)]],
                         rows[b], gsem[b])

    def _wait_scatter(b):
        pltpu.make_async_copy(rows[b], accum.at[dst_idx[b]], ssem[b]).wait()

    def _step(c, b, first=False):
        pltpu.make_async_copy(dst_hbm.at[pl.ds(0, CHUNK)],
                              dst_idx[b], dsem[b]).wait()
        pltpu.make_async_copy(ew_hbm.at[pl.ds(0, CHUNK)],
                              wts[b], dsem[b]).wait()
        pltpu.make_async_copy(pre_hbm.at[pl.ds(0, CHUNK)],
                              rows[b], gsem[b]).wait()

        def _scale(k8, _2):
            base = k8 * 8
            ws = [plsc.load_gather(wts[b],
                                   [jnp.full((16,), base + u, jnp.int32)])
                  for u in range(8)]
            for u in range(8):
                k = base + u
                for j in range(D_OUT // 16):
                    rows[b][k, pl.ds(j * 16, 16)] = (
                        rows[b][k, pl.ds(j * 16, 16)] * ws[u])
            return _2
        lax.fori_loop(0, CHUNK // 8, _scale, 0)
        pltpu.async_copy(rows[b], accum.at[dst_idx[b]], ssem[b], add=True)

        b2 = (b + 2) % NBUF
        if not isinstance(c, int) or c + 2 < NCHUNK:
            if not first:
                _wait_scatter(b2)
            _fire(c + 2, b2)

    _fire(0, 0)
    _fire(1, 1)
    _step(0, 0, first=True)

    def _trip(i, _):
        c = 3 * i + 1
        _step(c, 1)
        _step(c + 1, 2)
        _step(c + 2, 0)
        return _
    NTRIP = (NCHUNK - 5) // 3
    lax.fori_loop(0, NTRIP, _trip, 0)

    for c in range(3 * NTRIP + 1, NCHUNK):
        _step(c, c % NBUF)
    for b in range(NBUF):
        _wait_scatter(b)
    plsc.subcore_barrier()

    pltpu.sync_copy(
        accum.at[pl.ds(sid * ROWS_PER_TILE, ROWS_PER_TILE)],
        part_hbm.at[cid, pl.ds(sid * ROWS_PER_TILE, ROWS_PER_TILE)])


_sc_gcn = functools.partial(
    pl.kernel,
    out_type=jax.ShapeDtypeStruct((NC, N_PAD, D_OUT), jnp.float32),
    mesh=plsc.VectorSubcoreMesh(core_axis_name="c", subcore_axis_name="s"),
    scratch_types=(
        [pltpu.VMEM_SHARED((N_PAD, D_OUT), jnp.float32)]
        + [pltpu.VMEM((EPW,), jnp.int32)]
        + [pltpu.VMEM((CHUNK,), jnp.int32)] * NBUF
        + [pltpu.VMEM((CHUNK,), jnp.float32)] * NBUF
        + [pltpu.VMEM((CHUNK, D_OUT), jnp.float32)] * NBUF
        + [pltpu.VMEM((ZROWS, D_OUT), jnp.float32)]
        + [pltpu.SemaphoreType.DMA] * (3 * NBUF)
    ),
    compiler_params=pltpu.CompilerParams(needs_layout_passes=False),
)(_sc_body)


def _combine_body(p_ref, o_ref):
    o_ref[...] = jnp.maximum(p_ref[0] + p_ref[1], 0.0)


def _combine(part):
    return pl.pallas_call(
        _combine_body,
        out_shape=jax.ShapeDtypeStruct((N, D_OUT), jnp.float32),
        grid=(10,),
        in_specs=[pl.BlockSpec((NC, N // 10, D_OUT), lambda i: (0, i, 0))],
        out_specs=pl.BlockSpec((N // 10, D_OUT), lambda i: (i, 0)),
    )(part)


def kernel(x, W, edge_weight, edge_index):
    src = edge_index[0].astype(jnp.int32)
    dst = edge_index[1].astype(jnp.int32)
    pre_sup = _matmul(x, W)
    part = _sc_gcn(pre_sup, src, dst, edge_weight)
    return _combine(part)
